# baseline (speedup 1.0000x reference)
import functools

import jax
import jax.numpy as jnp
from jax import lax
from jax.experimental import pallas as pl
from jax.experimental.pallas import tpu as pltpu
from jax.experimental.pallas import tpu_sc as plsc

_M = 0.2
_S = 30.0
_LOG2E = 1.4426950408889634
_K1 = _S * _LOG2E

_W = 512
_G = 7
_K = 22
_SC_END = 99840


def _sc_gather_targets(inputs, targets):
    b, c = inputs.shape
    info = plsc.get_sparse_core_info()
    nw = info.num_cores * info.num_subcores
    bpw = b // nw
    mesh = plsc.VectorSubcoreMesh(core_axis_name="c", subcore_axis_name="s")

    @functools.partial(
        pl.kernel,
        mesh=mesh,
        out_type=jax.ShapeDtypeStruct((b,), jnp.float32),
        scratch_types=[
            pltpu.VMEM((bpw,), jnp.int32),
            pltpu.VMEM((bpw, 8, 128), jnp.float32),
            pltpu.VMEM((bpw,), jnp.float32),
            pltpu.SemaphoreType.DMA,
        ],
    )
    def gather(in_hbm, tgt_hbm, out_hbm, idx_v, tiles_v, xts_v, sem):
        wid = lax.axis_index("s") * info.num_cores + lax.axis_index("c")
        base = pl.multiple_of(wid * bpw, bpw)
        pltpu.sync_copy(tgt_hbm.at[pl.ds(base, bpw)], idx_v)
        lanes = lax.iota(jnp.int32, 16)
        ts, handles = [], []
        for chunk in range(bpw // 16):
            tv = idx_v[pl.ds(chunk * 16, 16)]
            for l in range(16):
                k = chunk * 16 + l
                t = tv[l]
                ts.append(t)
                cb = pl.multiple_of(jnp.bitwise_and(t, jnp.int32(-128)), 128)
                rb = (k // 8) * 8
                handles.append(
                    pltpu.async_copy(
                        in_hbm.at[pl.ds(base + rb, 8), pl.ds(cb, 128)],
                        tiles_v.at[k], sem))
        for h in handles:
            h.wait()
        for chunk in range(bpw // 16):
            xt_acc = jnp.zeros((16,), jnp.float32)
            for l in range(16):
                k = chunk * 16 + l
                lane = jnp.bitwise_and(ts[k], 127)
                sel = jnp.zeros((16,), jnp.float32)
                for l8 in range(8):
                    v = tiles_v[k, k % 8, pl.ds(l8 * 16, 16)]
                    sel = jnp.where(l8 * 16 + lanes == lane, v, sel)
                idxv = jnp.full((16,), jnp.bitwise_and(lane, 15), jnp.int32)
                v16 = lax.gather(
                    sel, idxv[:, None],
                    lax.GatherDimensionNumbers(
                        offset_dims=(), collapsed_slice_dims=(0,),
                        start_index_map=(0,)),
                    slice_sizes=(1,),
                    mode=lax.GatherScatterMode.PROMISE_IN_BOUNDS)
                xt_acc = jnp.where(lanes == l, v16, xt_acc)
            xts_v[pl.ds(chunk * 16, 16)] = xt_acc
        pltpu.sync_copy(xts_v, out_hbm.at[pl.ds(base, bpw)])

    return gather(inputs, targets)


def _sc_stream_partials(inputs, c0, c1):
    b, _ = inputs.shape
    info = plsc.get_sparse_core_info()
    nw = info.num_cores * info.num_subcores
    bpw = b // nw
    nrg = bpw // 8
    span = c1 - c0
    cw = 2048
    chunks = []
    off = c0
    while off < c1:
        clen = min(cw, c1 - off)
        chunks.append((off, clen))
        off += clen
    mesh = plsc.VectorSubcoreMesh(core_axis_name="c", subcore_axis_name="s")

    @functools.partial(
        pl.kernel,
        mesh=mesh,
        out_type=jax.ShapeDtypeStruct((b, 16), jnp.float32),
        scratch_types=[
            pltpu.VMEM((2, 8, cw), jnp.float32),
            pltpu.VMEM((bpw, 16), jnp.float32),
            pltpu.SemaphoreType.DMA,
            pltpu.SemaphoreType.DMA,
        ],
    )
    def stream(in_hbm, out_hbm, buf_v, part_v, sem0, sem1):
        wid = lax.axis_index("s") * info.num_cores + lax.axis_index("c")
        base = pl.multiple_of(wid * bpw, bpw)
        sems = (sem0, sem1)
        work = [(rg, off, clen) for rg in range(nrg)
                for (off, clen) in chunks]

        def start(i, slot):
            rg, off, clen = work[i]
            return pltpu.async_copy(
                in_hbm.at[pl.ds(base + rg * 8, 8), pl.ds(off, clen)],
                buf_v.at[slot, :, pl.ds(0, clen)], sems[slot])

        accs = [[jnp.zeros((16,), jnp.float32) for _ in range(8)]
                for _ in range(nrg)]
        pending = {0: start(0, 0)}
        for i, (rg, off, clen) in enumerate(work):
            slot = i % 2
            pending.pop(i).wait()
            if i + 1 < len(work):
                pending[i + 1] = start(i + 1, (i + 1) % 2)

            def body(it, a, slot=slot, clen=clen):
                cb = pl.multiple_of(it * 16, 16)
                out = []
                for r in range(8):
                    x = buf_v[slot, r, pl.ds(cb, 16)]
                    out.append(a[r] + jnp.exp(x * _S))
                return tuple(out)

            res = lax.fori_loop(0, clen // 16, body, tuple(accs[rg]))
            accs[rg] = list(res)
        for rg in range(nrg):
            for r in range(8):
                part_v[rg * 8 + r, :] = accs[rg][r]
        pltpu.sync_copy(part_v, out_hbm.at[pl.ds(base, bpw)])

    return stream(inputs)


def _tc_body(ng, c, *refs):
    x_refs = refs[:_G]
    o_ref, acc = refs[_G:]
    nc = pl.cdiv(c, _W)
    j = pl.program_id(0)

    @pl.when(j == 0)
    def _():
        acc[...] = jnp.zeros_like(acc)

    @pl.when(j < ng - 1)
    def _():
        s = jnp.zeros_like(acc)
        for g in range(_G):
            e = jnp.exp2(x_refs[g][...] * _K1)
            s += jnp.sum(e, axis=1, keepdims=True)
        acc[...] += s

    @pl.when(j == ng - 1)
    def _():
        cols = (nc - 1) * _W + jax.lax.broadcasted_iota(jnp.int32, (1, _W), 1)
        e = jnp.exp2(x_refs[_G - 1][...] * _K1)
        e = jnp.where(cols < c, e, 0.0)
        o_ref[...] = acc[...] + jnp.sum(e, axis=1, keepdims=True)


def _combine_body(s_ref, p_ref, xt_ref, o_ref):
    s = s_ref[...] + jnp.sum(p_ref[...], axis=1, keepdims=True)
    xt = xt_ref[...]
    e_old = jnp.exp2(xt * _K1)
    e_new = jnp.exp2((xt - _M) * _K1)
    s_mod = s - e_old + e_new
    loss = jnp.log(s_mod) - _S * (xt - _M)
    o_ref[...] = jnp.mean(loss, keepdims=True)


def kernel(inputs, targets):
    b, c = inputs.shape
    nc = pl.cdiv(c, _W)
    ng = _K + 1
    xt = _sc_gather_targets(inputs, targets).reshape(b, 1)
    scp = _sc_stream_partials(inputs, _G * _K * _W, _SC_END)

    def imap(g):
        if g < _G - 1:
            return lambda j: (0, g * _K + jnp.minimum(j, _K - 1))
        return lambda j: (0, jnp.where(j == _K, nc - 1, g * _K + j))

    in_specs = [pl.BlockSpec((b, _W), imap(g)) for g in range(_G)]
    s_tc = pl.pallas_call(
        functools.partial(_tc_body, ng, c),
        grid=(ng,),
        in_specs=in_specs,
        out_specs=pl.BlockSpec((b, 1), lambda j: (0, 0)),
        out_shape=jax.ShapeDtypeStruct((b, 1), jnp.float32),
        scratch_shapes=[pltpu.VMEM((b, 1), jnp.float32)],
    )(*([inputs] * _G))
    out = pl.pallas_call(
        _combine_body,
        in_specs=[
            pl.BlockSpec((b, 1), lambda: (0, 0)),
            pl.BlockSpec((b, 16), lambda: (0, 0)),
            pl.BlockSpec((b, 1), lambda: (0, 0)),
        ],
        out_specs=pl.BlockSpec((1, 1), lambda: (0, 0)),
        out_shape=jax.ShapeDtypeStruct((1, 1), jnp.float32),
    )(s_tc, scp, xt)
    return out[0, 0]


# speedup vs baseline: 1.0006x; 1.0006x over previous
import functools

import jax
import jax.numpy as jnp
from jax import lax
from jax.experimental import pallas as pl
from jax.experimental.pallas import tpu as pltpu
from jax.experimental.pallas import tpu_sc as plsc
from jax.experimental.xla_metadata import set_xla_metadata

_M = 0.2
_S = 30.0
_LOG2E = 1.4426950408889634
_K1 = _S * _LOG2E

_W = 512
_G = 7
_K = 22
_SC_END = 99840


def _sc_gather_targets(inputs, targets):
    b, c = inputs.shape
    info = plsc.get_sparse_core_info()
    nw = info.num_cores * info.num_subcores
    bpw = b // nw
    mesh = plsc.VectorSubcoreMesh(core_axis_name="c", subcore_axis_name="s")

    @functools.partial(
        pl.kernel,
        mesh=mesh,
        out_type=jax.ShapeDtypeStruct((b,), jnp.float32),
        scratch_types=[
            pltpu.VMEM((bpw,), jnp.int32),
            pltpu.VMEM((bpw, 8, 128), jnp.float32),
            pltpu.VMEM((bpw,), jnp.float32),
            pltpu.SemaphoreType.DMA,
        ],
    )
    def gather(in_hbm, tgt_hbm, out_hbm, idx_v, tiles_v, xts_v, sem):
        wid = lax.axis_index("s") * info.num_cores + lax.axis_index("c")
        base = pl.multiple_of(wid * bpw, bpw)
        pltpu.sync_copy(tgt_hbm.at[pl.ds(base, bpw)], idx_v)
        lanes = lax.iota(jnp.int32, 16)
        ts, handles = [], []
        for chunk in range(bpw // 16):
            tv = idx_v[pl.ds(chunk * 16, 16)]
            for l in range(16):
                k = chunk * 16 + l
                t = tv[l]
                ts.append(t)
                cb = pl.multiple_of(jnp.bitwise_and(t, jnp.int32(-128)), 128)
                rb = (k // 8) * 8
                handles.append(
                    pltpu.async_copy(
                        in_hbm.at[pl.ds(base + rb, 8), pl.ds(cb, 128)],
                        tiles_v.at[k], sem))
        for h in handles:
            h.wait()
        for chunk in range(bpw // 16):
            xt_acc = jnp.zeros((16,), jnp.float32)
            for l in range(16):
                k = chunk * 16 + l
                lane = jnp.bitwise_and(ts[k], 127)
                sel = jnp.zeros((16,), jnp.float32)
                for l8 in range(8):
                    v = tiles_v[k, k % 8, pl.ds(l8 * 16, 16)]
                    sel = jnp.where(l8 * 16 + lanes == lane, v, sel)
                idxv = jnp.full((16,), jnp.bitwise_and(lane, 15), jnp.int32)
                v16 = lax.gather(
                    sel, idxv[:, None],
                    lax.GatherDimensionNumbers(
                        offset_dims=(), collapsed_slice_dims=(0,),
                        start_index_map=(0,)),
                    slice_sizes=(1,),
                    mode=lax.GatherScatterMode.PROMISE_IN_BOUNDS)
                xt_acc = jnp.where(lanes == l, v16, xt_acc)
            xts_v[pl.ds(chunk * 16, 16)] = xt_acc
        pltpu.sync_copy(xts_v, out_hbm.at[pl.ds(base, bpw)])

    return gather(inputs, targets)


def _sc_stream_partials(inputs, c0, c1):
    b, _ = inputs.shape
    info = plsc.get_sparse_core_info()
    nw = info.num_cores * info.num_subcores
    bpw = b // nw
    nrg = bpw // 8
    span = c1 - c0
    cw = 2048
    chunks = []
    off = c0
    while off < c1:
        clen = min(cw, c1 - off)
        chunks.append((off, clen))
        off += clen
    mesh = plsc.VectorSubcoreMesh(core_axis_name="c", subcore_axis_name="s")

    @functools.partial(
        pl.kernel,
        mesh=mesh,
        out_type=jax.ShapeDtypeStruct((b, 16), jnp.float32),
        scratch_types=[
            pltpu.VMEM((2, 8, cw), jnp.float32),
            pltpu.VMEM((bpw, 16), jnp.float32),
            pltpu.SemaphoreType.DMA,
            pltpu.SemaphoreType.DMA,
        ],
    )
    def stream(in_hbm, out_hbm, buf_v, part_v, sem0, sem1):
        wid = lax.axis_index("s") * info.num_cores + lax.axis_index("c")
        base = pl.multiple_of(wid * bpw, bpw)
        sems = (sem0, sem1)
        work = [(rg, off, clen) for rg in range(nrg)
                for (off, clen) in chunks]

        def start(i, slot):
            rg, off, clen = work[i]
            return pltpu.async_copy(
                in_hbm.at[pl.ds(base + rg * 8, 8), pl.ds(off, clen)],
                buf_v.at[slot, :, pl.ds(0, clen)], sems[slot])

        accs = [[jnp.zeros((16,), jnp.float32) for _ in range(8)]
                for _ in range(nrg)]
        pending = {0: start(0, 0)}
        for i, (rg, off, clen) in enumerate(work):
            slot = i % 2
            pending.pop(i).wait()
            if i + 1 < len(work):
                pending[i + 1] = start(i + 1, (i + 1) % 2)

            def body(it, a, slot=slot, clen=clen):
                cb = pl.multiple_of(it * 16, 16)
                out = []
                for r in range(8):
                    x = buf_v[slot, r, pl.ds(cb, 16)]
                    out.append(a[r] + jnp.exp(x * _S))
                return tuple(out)

            res = lax.fori_loop(0, clen // 16, body, tuple(accs[rg]))
            accs[rg] = list(res)
        for rg in range(nrg):
            for r in range(8):
                part_v[rg * 8 + r, :] = accs[rg][r]
        pltpu.sync_copy(part_v, out_hbm.at[pl.ds(base, bpw)])

    return stream(inputs)


def _tc_body(ng, c, *refs):
    x_refs = refs[:_G]
    o_ref, acc = refs[_G:]
    nc = pl.cdiv(c, _W)
    j = pl.program_id(0)

    @pl.when(j == 0)
    def _():
        acc[...] = jnp.zeros_like(acc)

    @pl.when(j < ng - 1)
    def _():
        s = jnp.zeros_like(acc)
        for g in range(_G):
            e = jnp.exp2(x_refs[g][...] * _K1)
            s += jnp.sum(e, axis=1, keepdims=True)
        acc[...] += s

    @pl.when(j == ng - 1)
    def _():
        cols = (nc - 1) * _W + jax.lax.broadcasted_iota(jnp.int32, (1, _W), 1)
        e = jnp.exp2(x_refs[_G - 1][...] * _K1)
        e = jnp.where(cols < c, e, 0.0)
        o_ref[...] = acc[...] + jnp.sum(e, axis=1, keepdims=True)


def _combine_body(s_ref, p_ref, xt_ref, o_ref):
    s = s_ref[...] + jnp.sum(p_ref[...], axis=1, keepdims=True)
    xt = xt_ref[...]
    e_old = jnp.exp2(xt * _K1)
    e_new = jnp.exp2((xt - _M) * _K1)
    s_mod = s - e_old + e_new
    loss = jnp.log(s_mod) - _S * (xt - _M)
    o_ref[...] = jnp.mean(loss, keepdims=True)


def kernel(inputs, targets):
    b, c = inputs.shape
    nc = pl.cdiv(c, _W)
    ng = _K + 1
    with set_xla_metadata(_scheduling_group_id="1"):
        xt = _sc_gather_targets(inputs, targets).reshape(b, 1)
        scp = _sc_stream_partials(inputs, _G * _K * _W, _SC_END)

    def imap(g):
        if g < _G - 1:
            return lambda j: (0, g * _K + jnp.minimum(j, _K - 1))
        return lambda j: (0, jnp.where(j == _K, nc - 1, g * _K + j))

    in_specs = [pl.BlockSpec((b, _W), imap(g)) for g in range(_G)]
    with set_xla_metadata(_scheduling_group_id="1"):
        s_tc = pl.pallas_call(
            functools.partial(_tc_body, ng, c),
            grid=(ng,),
            in_specs=in_specs,
            out_specs=pl.BlockSpec((b, 1), lambda j: (0, 0)),
            out_shape=jax.ShapeDtypeStruct((b, 1), jnp.float32),
            scratch_shapes=[pltpu.VMEM((b, 1), jnp.float32)],
        )(*([inputs] * _G))
    out = pl.pallas_call(
        _combine_body,
        in_specs=[
            pl.BlockSpec((b, 1), lambda: (0, 0)),
            pl.BlockSpec((b, 16), lambda: (0, 0)),
            pl.BlockSpec((b, 1), lambda: (0, 0)),
        ],
        out_specs=pl.BlockSpec((1, 1), lambda: (0, 0)),
        out_shape=jax.ShapeDtypeStruct((1, 1), jnp.float32),
    )(s_tc, scp, xt)
    return out[0, 0]
